# baseline (speedup 1.0000x reference)
import jax
import jax.numpy as jnp
from jax import lax
from jax.experimental import pallas as pl
from jax.experimental.pallas import tpu as pltpu

BN_EPS = 1e-5
VMEM_LIMIT = 48 * 1024 * 1024


def _largest_divisor_leq(n, target):
    for d in range(min(n, target), 0, -1):
        if n % d == 0:
            return d
    return 1


def _deconv_kernel(x_ref, w_ref, b_ref, o_ref):
    h, w = o_ref.shape[0], o_ref.shape[2]
    cq = o_ref.shape[-1]
    xv = x_ref[0].astype(w_ref.dtype)
    y = lax.dot_general(xv, w_ref[0], (((0,), (0,)), ((), ())),
                        preferred_element_type=jnp.float32)
    y = y + b_ref[...]
    for a in range(2):
        o_ref[:, a] = y[:, a * cq:(a + 1) * cq].reshape(
            h, w, cq).astype(o_ref.dtype)


def _conv3_rows(Mv, T, B, w_ref, TH, W2, Cout):
    L = TH * W2
    C = Mv.shape[-1]
    lhs0 = jnp.concatenate([T, Mv[:TH - 1]], axis=0)
    lhs2 = jnp.concatenate([Mv[1:], B], axis=0)
    y = jnp.dot(lhs0.reshape(L, C), w_ref[0],
                preferred_element_type=jnp.float32)
    y = y + jnp.dot(Mv.reshape(L, C), w_ref[1],
                    preferred_element_type=jnp.float32)
    y = y + jnp.dot(lhs2.reshape(L, C), w_ref[2],
                    preferred_element_type=jnp.float32)
    return y


def _combine_kw(y, TH, W2, Cout):
    y = y.reshape(TH, W2, 3 * Cout)
    zpad = jnp.zeros((TH, 1, Cout), jnp.float32)
    z = y[:, :, Cout:2 * Cout]
    z = z + jnp.concatenate([zpad, y[:, :W2 - 1, 0:Cout]], axis=1)
    z = z + jnp.concatenate([y[:, 1:, 2 * Cout:], zpad], axis=1)
    return z


def _make_conv1_kernel(TH, RB, W2, Cout):
    def body(um_ref, ut_ref, ub_ref, dm_ref, dt_ref, db_ref,
             w_ref, z_ref, s1_ref, s2_ref):
        rb = pl.program_id(0) % RB
        tkeep = rb > 0
        bkeep = rb < RB - 1
        Mv = jnp.concatenate([um_ref[0], dm_ref[0]], axis=2)
        T = jnp.concatenate([ut_ref[0], dt_ref[0]], axis=2)
        B = jnp.concatenate([ub_ref[0], db_ref[0]], axis=2)
        T = jnp.where(tkeep, T, jnp.zeros_like(T))
        B = jnp.where(bkeep, B, jnp.zeros_like(B))
        y = _conv3_rows(Mv, T, B, w_ref, TH, W2, Cout)
        z = _combine_kw(y, TH, W2, Cout)
        zf = z.reshape(TH * W2, Cout)
        s1_ref[0] = jnp.sum(zf, axis=0, keepdims=True)
        s2_ref[0] = jnp.sum(zf * zf, axis=0, keepdims=True)
        z_ref[0] = z.astype(z_ref.dtype)

    return body


def _make_conv2_kernel(TH, RB, W2, Cout):
    def body(xm_ref, xt_ref, xb_ref, w01_ref, w2_ref, sc_ref, sh_ref,
             z_ref, s1_ref, s2_ref):
        rb = pl.program_id(0) % RB
        tkeep = rb > 0
        bkeep = rb < RB - 1
        C = xm_ref.shape[-1]
        L = TH * W2
        sc = sc_ref[...].reshape(1, 1, C)
        sh = sh_ref[...].reshape(1, 1, C)

        def act(v):
            return jnp.maximum(v.astype(jnp.float32) * sc + sh,
                               0.0).astype(v.dtype)

        Mv = act(xm_ref[0])
        T = jnp.where(tkeep, act(xt_ref[0]), jnp.zeros_like(xt_ref[0]))
        B = jnp.where(bkeep, act(xb_ref[0]), jnp.zeros_like(xb_ref[0]))
        lhs0 = jnp.concatenate([T, Mv[:TH - 1]], axis=0)
        lhs2 = jnp.concatenate([Mv[1:], B], axis=0)
        lhs01 = jnp.concatenate([lhs0, Mv], axis=2)
        y = jnp.dot(lhs01.reshape(L, 2 * C), w01_ref[0],
                    preferred_element_type=jnp.float32)
        y = y + jnp.dot(lhs2.reshape(L, C), w2_ref[0],
                        preferred_element_type=jnp.float32)
        z = _combine_kw(y, TH, W2, Cout)
        zf = z.reshape(TH * W2, Cout)
        s1_ref[0] = jnp.sum(zf, axis=0, keepdims=True)
        s2_ref[0] = jnp.sum(zf * zf, axis=0, keepdims=True)
        z_ref[0] = z.astype(z_ref.dtype)

    return body


def _bn_relu_t_kernel(z_ref, sc_ref, sh_ref, o_ref):
    c = sc_ref.shape[-1]
    z = z_ref[0].astype(jnp.float32)
    a = jnp.maximum(z * sc_ref[...].reshape(1, 1, c)
                    + sh_ref[...].reshape(1, 1, c), 0.0)
    o_ref[0] = jnp.transpose(a, (2, 0, 1))


def _bn_finalize(ssum, ssq, gamma, beta, count):
    m = jnp.sum(ssum[:, 0, :], axis=0) / count
    v = jnp.maximum(jnp.sum(ssq[:, 0, :], axis=0) / count - m * m, 0.0)
    sc = gamma.astype(jnp.float32) * lax.rsqrt(v + BN_EPS)
    sh = beta.astype(jnp.float32) - m * sc
    c = sc.shape[0]
    return sc.reshape(1, c), sh.reshape(1, c)


def kernel(up_x, down_x, up_w, up_b, c1_w, c1_b, c1_g, c1_beta,
           c2_w, c2_b, c2_g, c2_beta):
    N, Ci, H, W = up_x.shape
    Cout = up_w.shape[1]
    Cd = down_x.shape[1]
    H2, W2 = 2 * H, 2 * W
    Cin1 = Cout + Cd
    dtc = jnp.bfloat16

    down = jnp.transpose(down_x, (0, 2, 3, 1)).astype(dtc)

    w_up = jnp.transpose(up_w.astype(jnp.float32),
                         (2, 0, 3, 1)).reshape(2, Ci, 2 * Cout)
    w_up = jnp.transpose(w_up, (1, 0, 2)).reshape(1, Ci, 4 * Cout).astype(dtc)
    b_up = jnp.tile(up_b.astype(jnp.float32), 4).reshape(1, 4 * Cout)
    w1 = jnp.transpose(c1_w.astype(jnp.float32),
                       (2, 1, 3, 0)).reshape(3, Cin1, 3 * Cout).astype(dtc)
    w2 = jnp.transpose(c2_w.astype(jnp.float32),
                       (2, 1, 3, 0)).reshape(3, Cout, 3 * Cout).astype(dtc)
    w2_01 = jnp.concatenate([w2[0], w2[1]], axis=0).reshape(
        1, 2 * Cout, 3 * Cout)
    w2_2 = w2[2].reshape(1, Cout, 3 * Cout)

    R = N * H
    x2 = up_x.reshape(N, Ci, H * W)
    y = pl.pallas_call(
        _deconv_kernel,
        grid=(N,),
        in_specs=[
            pl.BlockSpec((1, Ci, H * W), lambda i: (i, 0, 0)),
            pl.BlockSpec((1, Ci, 4 * Cout), lambda i: (0, 0, 0)),
            pl.BlockSpec((1, 4 * Cout), lambda i: (0, 0)),
        ],
        out_specs=pl.BlockSpec((H, 2, W, 2 * Cout), lambda i: (i, 0, 0, 0)),
        out_shape=jax.ShapeDtypeStruct((R, 2, W, 2 * Cout), dtc),
        compiler_params=pltpu.CompilerParams(
            dimension_semantics=("parallel",), vmem_limit_bytes=VMEM_LIMIT),
        cost_estimate=pl.CostEstimate(
            flops=2 * R * W * Ci * 4 * Cout, transcendentals=0,
            bytes_accessed=2 * (R * W * Ci + R * 4 * W * Cout)),
    )(x2, w_up, b_up)
    u = y.reshape(N, H, 2, W, 2, Cout).reshape(N, H2, W2, Cout)

    TH = _largest_divisor_leq(H2, 32)
    RB = H2 // TH
    G = N * RB
    conv_flops = 2 * 9 * N * H2 * W2 * Cin1 * Cout

    def _main(g):
        return (g // RB, g % RB, 0, 0)

    def _top(g):
        return (g // RB, jnp.maximum((g % RB) * TH - 1, 0), 0, 0)

    def _bot(g):
        return (g // RB, jnp.minimum((g % RB) * TH + TH, H2 - 1), 0, 0)

    def _halo_specs(C):
        return [
            pl.BlockSpec((1, TH, W2, C), _main),
            pl.BlockSpec((1, 1, W2, C), _top),
            pl.BlockSpec((1, 1, W2, C), _bot),
        ]

    z1, s1a, s2a = pl.pallas_call(
        _make_conv1_kernel(TH, RB, W2, Cout),
        grid=(G,),
        in_specs=_halo_specs(Cout) + _halo_specs(Cd) + [
            pl.BlockSpec((3, Cin1, 3 * Cout), lambda g: (0, 0, 0)),
        ],
        out_specs=(
            pl.BlockSpec((1, TH, W2, Cout), _main),
            pl.BlockSpec((1, 1, Cout), lambda g: (g, 0, 0)),
            pl.BlockSpec((1, 1, Cout), lambda g: (g, 0, 0)),
        ),
        out_shape=(
            jax.ShapeDtypeStruct((N, H2, W2, Cout), dtc),
            jax.ShapeDtypeStruct((G, 1, Cout), jnp.float32),
            jax.ShapeDtypeStruct((G, 1, Cout), jnp.float32),
        ),
        compiler_params=pltpu.CompilerParams(
            dimension_semantics=("parallel",), vmem_limit_bytes=VMEM_LIMIT),
        cost_estimate=pl.CostEstimate(
            flops=conv_flops, transcendentals=0,
            bytes_accessed=2 * (N * H2 * W2 * Cin1 + N * H2 * W2 * Cout)),
    )(u, u, u, down, down, down, w1)

    count = float(N * H2 * W2)
    sc1, sh1 = _bn_finalize(s1a, s2a, c1_g, c1_beta, count)

    z2, s1b, s2b = pl.pallas_call(
        _make_conv2_kernel(TH, RB, W2, Cout),
        grid=(G,),
        in_specs=_halo_specs(Cout) + [
            pl.BlockSpec((1, 2 * Cout, 3 * Cout), lambda g: (0, 0, 0)),
            pl.BlockSpec((1, Cout, 3 * Cout), lambda g: (0, 0, 0)),
            pl.BlockSpec((1, Cout), lambda g: (0, 0)),
            pl.BlockSpec((1, Cout), lambda g: (0, 0)),
        ],
        out_specs=(
            pl.BlockSpec((1, TH, W2, Cout), _main),
            pl.BlockSpec((1, 1, Cout), lambda g: (g, 0, 0)),
            pl.BlockSpec((1, 1, Cout), lambda g: (g, 0, 0)),
        ),
        out_shape=(
            jax.ShapeDtypeStruct((N, H2, W2, Cout), dtc),
            jax.ShapeDtypeStruct((G, 1, Cout), jnp.float32),
            jax.ShapeDtypeStruct((G, 1, Cout), jnp.float32),
        ),
        compiler_params=pltpu.CompilerParams(
            dimension_semantics=("parallel",), vmem_limit_bytes=VMEM_LIMIT),
        cost_estimate=pl.CostEstimate(
            flops=2 * 9 * N * H2 * W2 * Cout * Cout, transcendentals=0,
            bytes_accessed=2 * 2 * N * H2 * W2 * Cout),
    )(z1, z1, z1, w2_01, w2_2, sc1, sh1)

    sc2, sh2 = _bn_finalize(s1b, s2b, c2_g, c2_beta, count)

    out = pl.pallas_call(
        _bn_relu_t_kernel,
        grid=(G,),
        in_specs=[
            pl.BlockSpec((1, TH, W2, Cout), lambda g: (g // RB, g % RB, 0, 0)),
            pl.BlockSpec((1, Cout), lambda g: (0, 0)),
            pl.BlockSpec((1, Cout), lambda g: (0, 0)),
        ],
        out_specs=pl.BlockSpec((1, Cout, TH, W2),
                               lambda g: (g // RB, 0, g % RB, 0)),
        out_shape=jax.ShapeDtypeStruct((N, Cout, H2, W2), jnp.float32),
        compiler_params=pltpu.CompilerParams(
            dimension_semantics=("parallel",), vmem_limit_bytes=VMEM_LIMIT),
        cost_estimate=pl.CostEstimate(
            flops=2 * N * H2 * W2 * Cout, transcendentals=0,
            bytes_accessed=6 * N * H2 * W2 * Cout),
    )(z2, sc2, sh2)
    return out


# speedup vs baseline: 1.0460x; 1.0460x over previous
import jax
import jax.numpy as jnp
from jax import lax
from jax.experimental import pallas as pl
from jax.experimental.pallas import tpu as pltpu

BN_EPS = 1e-5
VMEM_LIMIT = 48 * 1024 * 1024


def _largest_divisor_leq(n, target):
    for d in range(min(n, target), 0, -1):
        if n % d == 0:
            return d
    return 1


def _deconv_kernel(x_ref, w_ref, b_ref, o_ref):
    tr, w, cin = x_ref.shape
    xf = x_ref[...].reshape(tr * w, cin)
    for a in range(2):
        y = jnp.dot(xf, w_ref[a], preferred_element_type=jnp.float32)
        y = y + b_ref[...]
        o_ref[:, a] = y.reshape(tr, w, y.shape[-1]).astype(o_ref.dtype)


def _conv3_rows(Mv, T, B, w_ref, TH, W2, Cout):
    L = TH * W2
    C = Mv.shape[-1]
    lhs0 = jnp.concatenate([T, Mv[:TH - 1]], axis=0)
    lhs2 = jnp.concatenate([Mv[1:], B], axis=0)
    y = jnp.dot(lhs0.reshape(L, C), w_ref[0],
                preferred_element_type=jnp.float32)
    y = y + jnp.dot(Mv.reshape(L, C), w_ref[1],
                    preferred_element_type=jnp.float32)
    y = y + jnp.dot(lhs2.reshape(L, C), w_ref[2],
                    preferred_element_type=jnp.float32)
    return y


def _combine_kw(y, TH, W2, Cout):
    y = y.reshape(TH, W2, 3 * Cout)
    zpad = jnp.zeros((TH, 1, Cout), jnp.float32)
    z = y[:, :, Cout:2 * Cout]
    z = z + jnp.concatenate([zpad, y[:, :W2 - 1, 0:Cout]], axis=1)
    z = z + jnp.concatenate([y[:, 1:, 2 * Cout:], zpad], axis=1)
    return z


def _make_conv1_kernel(TH, RB, W2, Cout):
    def body(um_ref, ut_ref, ub_ref, dm_ref, dt_ref, db_ref,
             w_ref, z_ref, s1_ref, s2_ref):
        rb = pl.program_id(0) % RB
        tkeep = rb > 0
        bkeep = rb < RB - 1
        Mv = jnp.concatenate([um_ref[0], dm_ref[0]], axis=2)
        T = jnp.concatenate([ut_ref[0], dt_ref[0]], axis=2)
        B = jnp.concatenate([ub_ref[0], db_ref[0]], axis=2)
        T = jnp.where(tkeep, T, jnp.zeros_like(T))
        B = jnp.where(bkeep, B, jnp.zeros_like(B))
        y = _conv3_rows(Mv, T, B, w_ref, TH, W2, Cout)
        z = _combine_kw(y, TH, W2, Cout)
        zf = z.reshape(TH * W2, Cout)
        s1_ref[0] = jnp.sum(zf, axis=0, keepdims=True)
        s2_ref[0] = jnp.sum(zf * zf, axis=0, keepdims=True)
        z_ref[0] = z.astype(z_ref.dtype)

    return body


def _make_conv2_kernel(TH, RB, W2, Cout):
    def body(xm_ref, xt_ref, xb_ref, w01_ref, w2_ref, sc_ref, sh_ref,
             z_ref, s1_ref, s2_ref):
        rb = pl.program_id(0) % RB
        tkeep = rb > 0
        bkeep = rb < RB - 1
        C = xm_ref.shape[-1]
        L = TH * W2
        sc = sc_ref[...].reshape(1, 1, C)
        sh = sh_ref[...].reshape(1, 1, C)

        def act(v):
            return jnp.maximum(v.astype(jnp.float32) * sc + sh,
                               0.0).astype(v.dtype)

        Mv = act(xm_ref[0])
        T = jnp.where(tkeep, act(xt_ref[0]), jnp.zeros_like(xt_ref[0]))
        B = jnp.where(bkeep, act(xb_ref[0]), jnp.zeros_like(xb_ref[0]))
        lhs0 = jnp.concatenate([T, Mv[:TH - 1]], axis=0)
        lhs2 = jnp.concatenate([Mv[1:], B], axis=0)
        lhs01 = jnp.concatenate([lhs0, Mv], axis=2)
        y = jnp.dot(lhs01.reshape(L, 2 * C), w01_ref[0],
                    preferred_element_type=jnp.float32)
        y = y + jnp.dot(lhs2.reshape(L, C), w2_ref[0],
                        preferred_element_type=jnp.float32)
        z = _combine_kw(y, TH, W2, Cout)
        zf = z.reshape(TH * W2, Cout)
        s1_ref[0] = jnp.sum(zf, axis=0, keepdims=True)
        s2_ref[0] = jnp.sum(zf * zf, axis=0, keepdims=True)
        z_ref[0] = z.astype(z_ref.dtype)

    return body


def _bn_relu_t_kernel(z_ref, sc_ref, sh_ref, o_ref):
    c = sc_ref.shape[-1]
    z = z_ref[0].astype(jnp.float32)
    a = jnp.maximum(z * sc_ref[...].reshape(1, 1, c)
                    + sh_ref[...].reshape(1, 1, c), 0.0)
    o_ref[0] = jnp.transpose(a, (2, 0, 1))


def _bn_finalize(ssum, ssq, gamma, beta, count):
    m = jnp.sum(ssum[:, 0, :], axis=0) / count
    v = jnp.maximum(jnp.sum(ssq[:, 0, :], axis=0) / count - m * m, 0.0)
    sc = gamma.astype(jnp.float32) * lax.rsqrt(v + BN_EPS)
    sh = beta.astype(jnp.float32) - m * sc
    c = sc.shape[0]
    return sc.reshape(1, c), sh.reshape(1, c)


def kernel(up_x, down_x, up_w, up_b, c1_w, c1_b, c1_g, c1_beta,
           c2_w, c2_b, c2_g, c2_beta):
    N, Ci, H, W = up_x.shape
    Cout = up_w.shape[1]
    Cd = down_x.shape[1]
    H2, W2 = 2 * H, 2 * W
    Cin1 = Cout + Cd
    dtc = jnp.bfloat16

    up = jnp.transpose(up_x, (0, 2, 3, 1)).astype(dtc)
    down = jnp.transpose(down_x, (0, 2, 3, 1)).astype(dtc)

    w_up = jnp.transpose(up_w.astype(jnp.float32),
                         (2, 0, 3, 1)).reshape(2, Ci, 2 * Cout).astype(dtc)
    b_up = jnp.tile(up_b.astype(jnp.float32), 2).reshape(1, 2 * Cout)
    w1 = jnp.transpose(c1_w.astype(jnp.float32),
                       (2, 1, 3, 0)).reshape(3, Cin1, 3 * Cout).astype(dtc)
    w2 = jnp.transpose(c2_w.astype(jnp.float32),
                       (2, 1, 3, 0)).reshape(3, Cout, 3 * Cout).astype(dtc)
    w2_01 = jnp.concatenate([w2[0], w2[1]], axis=0).reshape(
        1, 2 * Cout, 3 * Cout)
    w2_2 = w2[2].reshape(1, Cout, 3 * Cout)

    R = N * H
    TR = _largest_divisor_leq(R, 64)
    x2 = up.reshape(R, W, Ci)
    y = pl.pallas_call(
        _deconv_kernel,
        grid=(R // TR,),
        in_specs=[
            pl.BlockSpec((TR, W, Ci), lambda i: (i, 0, 0)),
            pl.BlockSpec((2, Ci, 2 * Cout), lambda i: (0, 0, 0)),
            pl.BlockSpec((1, 2 * Cout), lambda i: (0, 0)),
        ],
        out_specs=pl.BlockSpec((TR, 2, W, 2 * Cout), lambda i: (i, 0, 0, 0)),
        out_shape=jax.ShapeDtypeStruct((R, 2, W, 2 * Cout), dtc),
        compiler_params=pltpu.CompilerParams(
            dimension_semantics=("parallel",), vmem_limit_bytes=VMEM_LIMIT),
        cost_estimate=pl.CostEstimate(
            flops=2 * R * W * Ci * 4 * Cout, transcendentals=0,
            bytes_accessed=2 * (R * W * Ci + R * 4 * W * Cout)),
    )(x2, w_up, b_up)
    u = y.reshape(N, H, 2, W, 2, Cout).reshape(N, H2, W2, Cout)

    TH = _largest_divisor_leq(H2, 32)
    RB = H2 // TH
    G = N * RB
    conv_flops = 2 * 9 * N * H2 * W2 * Cin1 * Cout

    def _main(g):
        return (g // RB, g % RB, 0, 0)

    def _top(g):
        return (g // RB, jnp.maximum((g % RB) * TH - 1, 0), 0, 0)

    def _bot(g):
        return (g // RB, jnp.minimum((g % RB) * TH + TH, H2 - 1), 0, 0)

    def _halo_specs(C):
        return [
            pl.BlockSpec((1, TH, W2, C), _main),
            pl.BlockSpec((1, 1, W2, C), _top),
            pl.BlockSpec((1, 1, W2, C), _bot),
        ]

    z1, s1a, s2a = pl.pallas_call(
        _make_conv1_kernel(TH, RB, W2, Cout),
        grid=(G,),
        in_specs=_halo_specs(Cout) + _halo_specs(Cd) + [
            pl.BlockSpec((3, Cin1, 3 * Cout), lambda g: (0, 0, 0)),
        ],
        out_specs=(
            pl.BlockSpec((1, TH, W2, Cout), _main),
            pl.BlockSpec((1, 1, Cout), lambda g: (g, 0, 0)),
            pl.BlockSpec((1, 1, Cout), lambda g: (g, 0, 0)),
        ),
        out_shape=(
            jax.ShapeDtypeStruct((N, H2, W2, Cout), dtc),
            jax.ShapeDtypeStruct((G, 1, Cout), jnp.float32),
            jax.ShapeDtypeStruct((G, 1, Cout), jnp.float32),
        ),
        compiler_params=pltpu.CompilerParams(
            dimension_semantics=("parallel",), vmem_limit_bytes=VMEM_LIMIT),
        cost_estimate=pl.CostEstimate(
            flops=conv_flops, transcendentals=0,
            bytes_accessed=2 * (N * H2 * W2 * Cin1 + N * H2 * W2 * Cout)),
    )(u, u, u, down, down, down, w1)

    count = float(N * H2 * W2)
    sc1, sh1 = _bn_finalize(s1a, s2a, c1_g, c1_beta, count)

    z2, s1b, s2b = pl.pallas_call(
        _make_conv2_kernel(TH, RB, W2, Cout),
        grid=(G,),
        in_specs=_halo_specs(Cout) + [
            pl.BlockSpec((1, 2 * Cout, 3 * Cout), lambda g: (0, 0, 0)),
            pl.BlockSpec((1, Cout, 3 * Cout), lambda g: (0, 0, 0)),
            pl.BlockSpec((1, Cout), lambda g: (0, 0)),
            pl.BlockSpec((1, Cout), lambda g: (0, 0)),
        ],
        out_specs=(
            pl.BlockSpec((1, TH, W2, Cout), _main),
            pl.BlockSpec((1, 1, Cout), lambda g: (g, 0, 0)),
            pl.BlockSpec((1, 1, Cout), lambda g: (g, 0, 0)),
        ),
        out_shape=(
            jax.ShapeDtypeStruct((N, H2, W2, Cout), dtc),
            jax.ShapeDtypeStruct((G, 1, Cout), jnp.float32),
            jax.ShapeDtypeStruct((G, 1, Cout), jnp.float32),
        ),
        compiler_params=pltpu.CompilerParams(
            dimension_semantics=("parallel",), vmem_limit_bytes=VMEM_LIMIT),
        cost_estimate=pl.CostEstimate(
            flops=2 * 9 * N * H2 * W2 * Cout * Cout, transcendentals=0,
            bytes_accessed=2 * 2 * N * H2 * W2 * Cout),
    )(z1, z1, z1, w2_01, w2_2, sc1, sh1)

    sc2, sh2 = _bn_finalize(s1b, s2b, c2_g, c2_beta, count)

    out = pl.pallas_call(
        _bn_relu_t_kernel,
        grid=(G,),
        in_specs=[
            pl.BlockSpec((1, TH, W2, Cout), lambda g: (g // RB, g % RB, 0, 0)),
            pl.BlockSpec((1, Cout), lambda g: (0, 0)),
            pl.BlockSpec((1, Cout), lambda g: (0, 0)),
        ],
        out_specs=pl.BlockSpec((1, Cout, TH, W2),
                               lambda g: (g // RB, 0, g % RB, 0)),
        out_shape=jax.ShapeDtypeStruct((N, Cout, H2, W2), jnp.float32),
        compiler_params=pltpu.CompilerParams(
            dimension_semantics=("parallel",), vmem_limit_bytes=VMEM_LIMIT),
        cost_estimate=pl.CostEstimate(
            flops=2 * N * H2 * W2 * Cout, transcendentals=0,
            bytes_accessed=6 * N * H2 * W2 * Cout),
    )(z2, sc2, sh2)
    return out
